# Initial kernel scaffold; baseline (speedup 1.0000x reference)
import jax
import jax.numpy as jnp
from jax.experimental import pallas as pl


def kernel(x):
    raise NotImplementedError("write your pallas kernel here")


# speedup vs baseline: 26.7294x; 26.7294x over previous
import functools

import numpy as np
import jax
import jax.numpy as jnp
from jax.experimental import pallas as pl

K_SEL = 32


def _cmpex_desc(v, j):
    n = v.shape[0]
    g = n // (2 * j)
    w = v.reshape(g, 2, j, *v.shape[1:])
    a = w[:, 0]
    b = w[:, 1]
    hi = jnp.maximum(a, b)
    lo = jnp.minimum(a, b)
    return jnp.concatenate([hi[:, None], lo[:, None]], axis=1).reshape(v.shape)


def _axis0_sign(n, k, dtype):
    i = jax.lax.broadcasted_iota(jnp.int32, (n, 1, 1), 0)
    return jnp.where((i & k) == 0, jnp.asarray(1.0, dtype),
                     jnp.asarray(-1.0, dtype))


def _sort32_desc(v):
    n = v.shape[0]
    k = 2
    while k <= n:
        if k < n:
            sgn = _axis0_sign(n, k, v.dtype)
            v = v * sgn
        j = k // 2
        while j >= 1:
            v = _cmpex_desc(v, j)
            j //= 2
        if k < n:
            v = v * sgn
        k *= 2
    return v


def _col_sign(w, dtype):
    c = jax.lax.broadcasted_iota(jnp.int32, (1, w, 1), 1)
    return jnp.where(c < w // 2, jnp.asarray(1.0, dtype),
                     jnp.asarray(-1.0, dtype))


def _topk_kernel(x_ref, o_ref):
    v = x_ref[0]
    c = v.shape[1]
    v = _sort32_desc(v * _col_sign(c, v.dtype))
    w = c
    while w > 1:
        a = v[:, : w // 2]
        b = v[:, w // 2:]
        m = jnp.maximum(a, -b)
        wout = w // 2
        if wout > 1:
            m = m * _col_sign(wout, v.dtype)
        j = K_SEL // 2
        while j >= 1:
            m = _cmpex_desc(m, j)
            j //= 2
        v = m
        w = wout
    o_ref[0] = v[:, 0, :]


@jax.jit
def kernel(x):
    b, l, d = x.shape
    k = K_SEL
    cols = l // k
    xr = x.reshape(b, k, cols, d)
    dblk = 256
    grid = (b, d // dblk)
    out = pl.pallas_call(
        _topk_kernel,
        grid=grid,
        in_specs=[
            pl.BlockSpec((1, k, cols, dblk), lambda i, j: (i, 0, 0, j)),
        ],
        out_specs=pl.BlockSpec((1, k, dblk), lambda i, j: (i, 0, j)),
        out_shape=jax.ShapeDtypeStruct((b, k, d), x.dtype),
    )(xr)
    return jnp.swapaxes(out, 1, 2).reshape(b, d * k)


# speedup vs baseline: 28.4383x; 1.0639x over previous
import jax
import jax.numpy as jnp
from jax.experimental import pallas as pl

K_SEL = 32


def _cmpex_desc(v, j):
    n = v.shape[0]
    g = n // (2 * j)
    w = v.reshape(g, 2, j, *v.shape[1:])
    a = w[:, 0]
    b = w[:, 1]
    hi = jnp.maximum(a, b)
    lo = jnp.minimum(a, b)
    return jnp.concatenate([hi[:, None], lo[:, None]], axis=1).reshape(v.shape)


def _cmpex_bidir(v, k, j):
    n = v.shape[0]
    g2 = n // (2 * k)
    kj = k // (2 * j)
    w = v.reshape(g2, 2, kj, 2, j, *v.shape[1:])
    a0, b0 = w[:, 0, :, 0], w[:, 0, :, 1]
    a1, b1 = w[:, 1, :, 0], w[:, 1, :, 1]
    r0 = jnp.concatenate(
        [jnp.maximum(a0, b0)[:, :, None], jnp.minimum(a0, b0)[:, :, None]],
        axis=2)
    r1 = jnp.concatenate(
        [jnp.minimum(a1, b1)[:, :, None], jnp.maximum(a1, b1)[:, :, None]],
        axis=2)
    return jnp.concatenate([r0[:, None], r1[:, None]], axis=1).reshape(v.shape)


def _sort32_desc(v):
    n = v.shape[0]
    k = 2
    while k < n:
        j = k // 2
        while j >= 1:
            v = _cmpex_bidir(v, k, j)
            j //= 2
        k *= 2
    j = n // 2
    while j >= 1:
        v = _cmpex_desc(v, j)
        j //= 2
    return v


def _col_sign(w, dtype):
    c = jax.lax.broadcasted_iota(jnp.int32, (1, w, 1), 1)
    return jnp.where(c < w // 2, jnp.asarray(1.0, dtype),
                     jnp.asarray(-1.0, dtype))


def _topk_kernel(x_ref, o_ref):
    v = x_ref[0]
    c = v.shape[1]
    v = _sort32_desc(v * _col_sign(c, v.dtype))
    w = c
    while w > 1:
        wo = w // 2
        if wo > 1:
            h = wo // 2
            a1 = v[:, :h]
            a2 = v[:, h:wo]
            b1 = v[:, wo:wo + h]
            b2 = v[:, wo + h:]
            m = jnp.concatenate(
                [jnp.maximum(a1, -b1), jnp.minimum(-a2, b2)], axis=1)
        else:
            m = jnp.maximum(v[:, :1], -v[:, 1:])
        j = K_SEL // 2
        while j >= 1:
            m = _cmpex_desc(m, j)
            j //= 2
        v = m
        w = wo
    o_ref[0] = v[:, 0, :]


@jax.jit
def kernel(x):
    b, l, d = x.shape
    k = K_SEL
    cols = l // k
    xr = x.reshape(b, k, cols, d)
    dblk = 256
    grid = (b, d // dblk)
    out = pl.pallas_call(
        _topk_kernel,
        grid=grid,
        in_specs=[
            pl.BlockSpec((1, k, cols, dblk), lambda i, j: (i, 0, 0, j)),
        ],
        out_specs=pl.BlockSpec((1, k, dblk), lambda i, j: (i, 0, j)),
        out_shape=jax.ShapeDtypeStruct((b, k, d), x.dtype),
    )(xr)
    return jnp.swapaxes(out, 1, 2).reshape(b, d * k)


# speedup vs baseline: 45.0105x; 1.5827x over previous
import jax
import jax.numpy as jnp
from jax.experimental import pallas as pl

K_SEL = 32
SLAB_W = 16
DBLK = 256


def _pair_stage(lists, k, j, asc):
    n = len(lists)
    for i in range(n):
        if i & j:
            continue
        p = i | j
        nhi, nlo = [], []
        for a, b in zip(lists[i], lists[p]):
            nhi.append(jnp.maximum(a, b))
            nlo.append(jnp.minimum(a, b))
        if ((i & k) == 0) != asc:
            lists[i], lists[p] = nhi, nlo
        else:
            lists[i], lists[p] = nlo, nhi


def _sort32(lists, asc):
    n = len(lists)
    k = 2
    while k < n:
        j = k // 2
        while j >= 1:
            _pair_stage(lists, k, j, asc)
            j //= 2
        k *= 2
    j = n // 2
    while j >= 1:
        _pair_stage(lists, n, j, asc)
        j //= 2


def _merge_net(lists, asc):
    n = len(lists)
    j = n // 2
    while j >= 1:
        _pair_stage(lists, n, j, asc)
        j //= 2


def _halves(slabs):
    if len(slabs) >= 2:
        mid = len(slabs) // 2
        return slabs[:mid], slabs[mid:]
    a = slabs[0]
    h = a.shape[0] // 2
    return [a[:h]], [a[h:]]


def _topk_kernel(x_ref, o_ref):
    n = K_SEL
    cols = x_ref.shape[2]
    half = cols // 2
    nslab = max(1, half // SLAB_W)
    sw = half // nslab
    p = [[x_ref[0, i, s * sw:(s + 1) * sw, :] for s in range(nslab)]
         for i in range(n)]
    m = [[x_ref[0, i, half + s * sw:half + (s + 1) * sw, :]
          for s in range(nslab)] for i in range(n)]
    _sort32(p, asc=False)
    _sort32(m, asc=True)
    w = half
    while w > 1:
        np_, nn_ = [], []
        for i in range(n):
            pa, pb = _halves(p[i])
            ma, mb = _halves(m[i])
            np_.append([jnp.maximum(a, b) for a, b in zip(pa, ma)])
            nn_.append([jnp.maximum(a, b) for a, b in zip(pb, mb)])
        _merge_net(np_, asc=False)
        _merge_net(nn_, asc=True)
        p, m = np_, nn_
        w //= 2
    flist = [[jnp.maximum(p[i][0], m[i][0])] for i in range(n)]
    _merge_net(flist, asc=False)
    o_ref[0] = jnp.concatenate([f[0] for f in flist], axis=0)


@jax.jit
def kernel(x):
    b, l, d = x.shape
    k = K_SEL
    cols = l // k
    xr = x.reshape(b, k, cols, d)
    dblk = DBLK
    grid = (b, d // dblk)
    out = pl.pallas_call(
        _topk_kernel,
        grid=grid,
        in_specs=[
            pl.BlockSpec((1, k, cols, dblk), lambda i, j: (i, 0, 0, j)),
        ],
        out_specs=pl.BlockSpec((1, k, dblk), lambda i, j: (i, 0, j)),
        out_shape=jax.ShapeDtypeStruct((b, k, d), x.dtype),
    )(xr)
    return jnp.swapaxes(out, 1, 2).reshape(b, d * k)


# speedup vs baseline: 63.5483x; 1.4119x over previous
import jax
import jax.numpy as jnp
from jax.experimental import pallas as pl

K_SEL = 32
SLAB_W = 8
DBLK = 256


def _pair_stage(col, k, j, asc):
    n = len(col)
    for i in range(n):
        if i & j:
            continue
        p = i | j
        a, b = col[i], col[p]
        hi = jnp.maximum(a, b)
        lo = jnp.minimum(a, b)
        if ((i & k) == 0) != asc:
            col[i], col[p] = hi, lo
        else:
            col[i], col[p] = lo, hi


def _sort32(col, asc):
    n = len(col)
    k = 2
    while k < n:
        j = k // 2
        while j >= 1:
            _pair_stage(col, k, j, asc)
            j //= 2
        k *= 2
    j = n // 2
    while j >= 1:
        _pair_stage(col, n, j, asc)
        j //= 2


def _merge_net(col, asc):
    n = len(col)
    j = n // 2
    while j >= 1:
        _pair_stage(col, n, j, asc)
        j //= 2


def _topk_kernel(x_ref, o_ref):
    n = K_SEL
    cols = x_ref.shape[2]
    nslab = cols // SLAB_W
    entries = []
    for s in range(nslab):
        col = [x_ref[0, i, s * SLAB_W:(s + 1) * SLAB_W, :] for i in range(n)]
        _sort32(col, asc=s >= nslab // 2)
        entries.append(col)
    width = SLAB_W
    while len(entries) > 1 or width > 1:
        half = len(entries) // 2
        entries = [[jnp.maximum(a, b)
                    for a, b in zip(entries[t], entries[half + t])]
                   for t in range(half)]
        if len(entries) == 1 and width > 1:
            h = width // 2
            top = [a[:h] for a in entries[0]]
            bot = [a[h:] for a in entries[0]]
            entries = [top, bot]
            width = h
        done = len(entries) == 1 and width == 1
        ne = len(entries)
        for t, e in enumerate(entries):
            _merge_net(e, asc=(not done) and t >= ne // 2)
    o_ref[0] = jnp.concatenate(entries[0], axis=0)


@jax.jit
def kernel(x):
    b, l, d = x.shape
    k = K_SEL
    cols = l // k
    xr = x.reshape(b, k, cols, d)
    dblk = DBLK
    grid = (b, d // dblk)
    out = pl.pallas_call(
        _topk_kernel,
        grid=grid,
        in_specs=[
            pl.BlockSpec((1, k, cols, dblk), lambda i, j: (i, 0, 0, j)),
        ],
        out_specs=pl.BlockSpec((1, k, dblk), lambda i, j: (i, 0, j)),
        out_shape=jax.ShapeDtypeStruct((b, k, d), x.dtype),
    )(xr)
    return jnp.swapaxes(out, 1, 2).reshape(b, d * k)
